# baseline (speedup 1.0000x reference)
import functools

import jax
import jax.numpy as jnp
from jax import lax
from jax.experimental import pallas as pl
from jax.experimental.pallas import tpu as pltpu
from jax.experimental.pallas import tpu_sc as plsc

_BETA = 1.0
_TEMPERATURE = 0.07
_K = 8
_BIAS_VAL = _BETA / _TEMPERATURE

_NW = 32
_ZCH = 65536


def _topk_idx_kernel(c_ref, mT_ref, arows_ref, idx_ref, *, R, M):
    c = c_ref[0:1, :]
    gx = -2.0 * mT_ref[0:1, :]
    gy = -2.0 * mT_ref[1:2, :]
    gz = -2.0 * mT_ref[2:3, :]
    a0 = arows_ref[:, 0:1]
    a1 = arows_ref[:, 1:2]
    a2 = arows_ref[:, 2:3]
    S = c + a0 * gx + a1 * gy + a2 * gz

    NCOL = M // 128
    cols = [S[:, k * 128:(k + 1) * 128] for k in range(NCOL)]
    INF = jnp.float32(jnp.inf)

    m0 = cols[0]
    k0 = jnp.zeros_like(m0, dtype=jnp.int32)
    m1 = jnp.full_like(m0, INF)
    m2 = m1
    m3 = m1
    k1 = k0
    k2 = k0
    k3 = k0
    for k in range(1, NCOL):
        v = cols[k]
        kv = jnp.full_like(k0, k)
        cnd = v < m0
        dv = jnp.where(cnd, m0, v)
        dk = jnp.where(cnd, k0, kv)
        m0 = jnp.where(cnd, v, m0)
        k0 = jnp.where(cnd, kv, k0)
        cnd = dv < m1
        dv2 = jnp.where(cnd, m1, dv)
        dk2 = jnp.where(cnd, k1, dk)
        m1 = jnp.where(cnd, dv, m1)
        k1 = jnp.where(cnd, dk, k1)
        cnd = dv2 < m2
        dv3 = jnp.where(cnd, m2, dv2)
        dk3 = jnp.where(cnd, k2, dk2)
        m2 = jnp.where(cnd, dv2, m2)
        k2 = jnp.where(cnd, dk2, k2)
        cnd = dv3 < m3
        m3 = jnp.where(cnd, dv3, m3)
        k3 = jnp.where(cnd, dk3, k3)

    lane = jax.lax.broadcasted_iota(jnp.int32, (R, 128), 1)
    BIG = jnp.int32(1 << 28)
    cur = m0
    cnt = jnp.zeros_like(k0)
    for t in range(_K):
        tau = jnp.min(cur, axis=1, keepdims=True)
        hit = cur == tau
        hl = jnp.min(jnp.where(hit, lane, 128), axis=1, keepdims=True)
        hit1 = lane == hl
        ksel = jnp.where(cnt == 0, k0,
                         jnp.where(cnt == 1, k1,
                                   jnp.where(cnt == 2, k2, k3)))
        kh = jnp.min(jnp.where(hit1, ksel, BIG), axis=1, keepdims=True)
        idx_ref[:, t:t + 1] = kh * 128 + hl
        cnt = cnt + hit1.astype(jnp.int32)
        nxt = jnp.where(cnt == 1, m1,
                        jnp.where(cnt == 2, m2,
                                  jnp.where(cnt == 3, m3, INF)))
        cur = jnp.where(hit1, nxt, cur)


def _make_sc_bias(TOT, NCH):
    SHARD = TOT // _NW
    NZ = SHARD // _ZCH
    mesh = plsc.VectorSubcoreMesh(core_axis_name="c", subcore_axis_name="s")

    @functools.partial(
        pl.kernel, mesh=mesh,
        out_type=jax.ShapeDtypeStruct((TOT,), jnp.float32),
        scratch_types=[
            pltpu.VMEM((_ZCH,), jnp.float32),
            pltpu.VMEM((NCH, 128), jnp.int32),
            pltpu.VMEM((128,), jnp.float32),
            pltpu.SemaphoreType.DMA,
        ],
    )
    def sc_bias(idx_hbm, out_hbm, zbuf, fidx, vals, sem):
        wid = lax.axis_index("s") * 2 + lax.axis_index("c")

        def fill16(i, ref, v):
            ref[pl.ds(i * 16, 16)] = jnp.full((16,), v)

        pl.loop(0, _ZCH // 16)(lambda i: fill16(i, zbuf, jnp.float32(0.0)))
        pl.loop(0, 8)(lambda i: fill16(i, vals, jnp.float32(_BIAS_VAL)))
        pltpu.sync_copy(idx_hbm.at[wid], fidx)
        base = wid * SHARD

        def zero_chunk(i):
            pltpu.sync_copy(zbuf, out_hbm.at[pl.ds(base + i * _ZCH, _ZCH)])

        pl.loop(0, NZ)(zero_chunk)

        def scatter_chunk(i):
            pltpu.async_copy(vals, out_hbm.at[fidx.at[i]], sem).wait()

        pl.loop(0, NCH)(scatter_chunk)

    return sc_bias


def kernel(anchors, n, d):
    B, M, _ = anchors.shape
    R = 512
    n_hat = n / (jnp.linalg.norm(n, axis=-1, keepdims=True) + 1e-8)
    s = jnp.einsum('bmc,bc->bm', anchors, n_hat) + d
    mirrored = anchors - 2.0 * s[..., None] * n_hat[:, None, :]
    mT = jnp.swapaxes(mirrored, 1, 2)
    c = jnp.sum(mirrored * mirrored, axis=-1)[:, None, :]

    body = functools.partial(_topk_idx_kernel, R=R, M=M)
    idx8 = pl.pallas_call(
        body,
        grid=(B, M // R),
        in_specs=[
            pl.BlockSpec((None, 1, M), lambda b, r: (b, 0, 0)),
            pl.BlockSpec((None, 3, M), lambda b, r: (b, 0, 0)),
            pl.BlockSpec((None, R, 3), lambda b, r: (b, r, 0)),
        ],
        out_specs=pl.BlockSpec((None, R, _K), lambda b, r: (b, r, 0)),
        out_shape=jax.ShapeDtypeStruct((B, M, _K), jnp.int32),
    )(c, mT, anchors)

    TOT = B * M * M
    NCH = (B * M * _K // _NW) // 128
    rows = jnp.arange(B * M, dtype=jnp.int32)[:, None]
    flat = (rows * M + idx8.reshape(B * M, _K)).reshape(_NW, NCH, 128)
    out = _make_sc_bias(TOT, NCH)(flat)
    return out.reshape(B, M, M)


# speedup vs baseline: 4.2771x; 4.2771x over previous
import functools

import jax
import jax.numpy as jnp
from jax.experimental import pallas as pl

_BETA = 1.0
_TEMPERATURE = 0.07
_K = 8
_BIAS_VAL = _BETA / _TEMPERATURE


def _bias_block_kernel(c_ref, mT_ref, arows_ref, out_ref, *, R, M):
    c = c_ref[0:1, :]
    gx = -2.0 * mT_ref[0:1, :]
    gy = -2.0 * mT_ref[1:2, :]
    gz = -2.0 * mT_ref[2:3, :]
    a0 = arows_ref[:, 0:1]
    a1 = arows_ref[:, 1:2]
    a2 = arows_ref[:, 2:3]
    S = c + a0 * gx + a1 * gy + a2 * gz

    NCOL = M // 128
    cols = [S[:, k * 128:(k + 1) * 128] for k in range(NCOL)]
    INF = jnp.float32(jnp.inf)

    m0 = cols[0]
    m1 = jnp.full_like(m0, INF)
    m2 = m1
    m3 = m1
    for k in range(1, NCOL):
        v = cols[k]
        t = jnp.maximum(m0, v)
        m0 = jnp.minimum(m0, v)
        v = t
        t = jnp.maximum(m1, v)
        m1 = jnp.minimum(m1, v)
        v = t
        t = jnp.maximum(m2, v)
        m2 = jnp.minimum(m2, v)
        m3 = jnp.minimum(m3, t)

    cur = m0
    cnt = jnp.zeros_like(m0, dtype=jnp.int32)
    tau = None
    for _ in range(_K):
        tau = jnp.min(cur, axis=1, keepdims=True)
        hit = cur == tau
        cnt = cnt + hit.astype(jnp.int32)
        nxt = jnp.where(cnt == 1, m1,
                        jnp.where(cnt == 2, m2,
                                  jnp.where(cnt == 3, m3, INF)))
        cur = jnp.where(hit, nxt, cur)

    count = None
    for k in range(NCOL):
        selk = cols[k] <= tau
        out_ref[:, k * 128:(k + 1) * 128] = jnp.where(
            selk, _BIAS_VAL, 0.0).astype(jnp.float32)
        ck = selk.astype(jnp.int32)
        count = ck if count is None else count + ck
    counts = jnp.sum(count, axis=1)
    bad = jnp.any(counts != _K)

    @pl.when(bad)
    def _exact_fallback():
        iota = jax.lax.broadcasted_iota(jnp.int32, (R, M), 1)
        Sf = S
        acc = jnp.zeros((R, M), dtype=jnp.bool_)
        for _ in range(_K):
            m = jnp.min(Sf, axis=1, keepdims=True)
            cand = jnp.where(Sf == m, iota, M)
            j = jnp.min(cand, axis=1, keepdims=True)
            hit = iota == j
            acc = jnp.logical_or(acc, hit)
            Sf = jnp.where(hit, jnp.inf, Sf)
        out_ref[...] = jnp.where(acc, _BIAS_VAL, 0.0).astype(jnp.float32)


def kernel(anchors, n, d):
    B, M, _ = anchors.shape
    R = 512
    n_hat = n / (jnp.linalg.norm(n, axis=-1, keepdims=True) + 1e-8)
    s = jnp.einsum('bmc,bc->bm', anchors, n_hat) + d
    mirrored = anchors - 2.0 * s[..., None] * n_hat[:, None, :]
    mT = jnp.swapaxes(mirrored, 1, 2)
    c = jnp.sum(mirrored * mirrored, axis=-1)[:, None, :]

    body = functools.partial(_bias_block_kernel, R=R, M=M)
    out = pl.pallas_call(
        body,
        grid=(B, M // R),
        in_specs=[
            pl.BlockSpec((None, 1, M), lambda b, r: (b, 0, 0)),
            pl.BlockSpec((None, 3, M), lambda b, r: (b, 0, 0)),
            pl.BlockSpec((None, R, 3), lambda b, r: (b, r, 0)),
        ],
        out_specs=pl.BlockSpec((None, R, M), lambda b, r: (b, r, 0)),
        out_shape=jax.ShapeDtypeStruct((B, M, M), jnp.float32),
    )(c, mT, anchors)
    return out
